# Initial kernel scaffold; baseline (speedup 1.0000x reference)
import jax
import jax.numpy as jnp
from jax.experimental import pallas as pl


def kernel(image, W1, b1, W2, b2):
    raise NotImplementedError("write your pallas kernel here")


# speedup vs baseline: 3.2975x; 3.2975x over previous
import jax
import jax.numpy as jnp
from jax.experimental import pallas as pl

_SLEN = 512
_PT = 16
_STEP = 8
_NB = _SLEN // _STEP
_NPD = (_SLEN - _PT) // _STEP + 1
_HID = 64
_ODIM = 32


def _enc_kernel(blocks_ref, q_ref, b1_ref, w2_ref, b2_ref, out_ref):
    blk = blocks_ref[0]
    p = jnp.dot(blk, q_ref[...], preferred_element_type=jnp.float32)
    a = p[:, 0:64]
    bq = p[:, 64:128]
    cq = p[:, 128:192]
    dq = p[:, 192:256]
    bs = jnp.roll(bq, -1, axis=0)
    cs = jnp.roll(cq, -64, axis=0)
    ds = jnp.roll(dq, -65, axis=0)
    h = jnp.maximum(a + bs + cs + ds + b1_ref[...], 0.0)
    out_ref[0] = jnp.dot(h, w2_ref[...], preferred_element_type=jnp.float32) + b2_ref[...]


def kernel(image, W1, b1, W2, b2):
    B = image.shape[0]
    blocks = (
        image.reshape(B, _NB, _STEP, _NB, _STEP)
        .transpose(0, 1, 3, 2, 4)
        .reshape(B, _NB * _NB, _STEP * _STEP)
    )
    w1r = W1.reshape(_PT, _PT, _HID)
    q = jnp.concatenate(
        [
            w1r[0:8, 0:8].reshape(64, _HID),
            w1r[0:8, 8:16].reshape(64, _HID),
            w1r[8:16, 0:8].reshape(64, _HID),
            w1r[8:16, 8:16].reshape(64, _HID),
        ],
        axis=1,
    )

    out_full = pl.pallas_call(
        _enc_kernel,
        grid=(B,),
        in_specs=[
            pl.BlockSpec((1, _NB * _NB, 64), lambda b: (b, 0, 0)),
            pl.BlockSpec((64, 4 * _HID), lambda b: (0, 0)),
            pl.BlockSpec((1, _HID), lambda b: (0, 0)),
            pl.BlockSpec((_HID, _ODIM), lambda b: (0, 0)),
            pl.BlockSpec((1, _ODIM), lambda b: (0, 0)),
        ],
        out_specs=pl.BlockSpec((1, _NB * _NB, _ODIM), lambda b: (b, 0, 0)),
        out_shape=jax.ShapeDtypeStruct((B, _NB * _NB, _ODIM), jnp.float32),
    )(blocks, q, b1.reshape(1, _HID), W2, b2.reshape(1, _ODIM))

    out = (
        out_full.reshape(B, _NB, _NB, _ODIM)[:, :_NPD, :_NPD, :]
        .reshape(B * _NPD * _NPD, _ODIM)
    )
    return out


# speedup vs baseline: 3.4471x; 1.0454x over previous
import jax
import jax.numpy as jnp
from jax.experimental import pallas as pl

_SLEN = 512
_PT = 16
_STEP = 8
_NB = _SLEN // _STEP
_NPD = (_SLEN - _PT) // _STEP + 1
_HID = 64
_ODIM = 32


def _enc_kernel(img_ref, q_ref, b1_ref, w2_ref, b2_ref, out_ref):
    img = img_ref[0, 0].astype(jnp.bfloat16)
    blk = (
        img.reshape(_NB, _STEP, _NB, _STEP)
        .transpose(0, 2, 1, 3)
        .reshape(_NB * _NB, _STEP * _STEP)
    )
    p = jnp.dot(blk, q_ref[...], preferred_element_type=jnp.float32)
    x = p[:, 0:64] + jnp.roll(p[:, 128:192], -64, axis=0)
    y = p[:, 64:128] + jnp.roll(p[:, 192:256], -64, axis=0)
    h = jnp.maximum(x + jnp.roll(y, -1, axis=0) + b1_ref[...], 0.0)
    o = jnp.dot(h.astype(jnp.bfloat16), w2_ref[...],
                preferred_element_type=jnp.float32) + b2_ref[...]
    out_ref[0] = o.reshape(_NB, _NB, _ODIM)[0:_NPD, 0:_NPD, :]


def kernel(image, W1, b1, W2, b2):
    B = image.shape[0]
    w1r = W1.reshape(_PT, _PT, _HID)
    q = jnp.concatenate(
        [
            w1r[0:8, 0:8].reshape(64, _HID),
            w1r[0:8, 8:16].reshape(64, _HID),
            w1r[8:16, 0:8].reshape(64, _HID),
            w1r[8:16, 8:16].reshape(64, _HID),
        ],
        axis=1,
    ).astype(jnp.bfloat16)

    out = pl.pallas_call(
        _enc_kernel,
        grid=(B,),
        in_specs=[
            pl.BlockSpec((1, 1, _SLEN, _SLEN), lambda b: (b, 0, 0, 0)),
            pl.BlockSpec((64, 4 * _HID), lambda b: (0, 0)),
            pl.BlockSpec((1, _HID), lambda b: (0, 0)),
            pl.BlockSpec((_HID, _ODIM), lambda b: (0, 0)),
            pl.BlockSpec((1, _ODIM), lambda b: (0, 0)),
        ],
        out_specs=pl.BlockSpec((1, _NPD, _NPD, _ODIM), lambda b: (b, 0, 0, 0)),
        out_shape=jax.ShapeDtypeStruct((B, _NPD, _NPD, _ODIM), jnp.float32),
    )(image, q, b1.reshape(1, _HID), W2.astype(jnp.bfloat16), b2.reshape(1, _ODIM))

    return out.reshape(B * _NPD * _NPD, _ODIM)


# speedup vs baseline: 3.7782x; 1.0960x over previous
import jax
import jax.numpy as jnp
from jax.experimental import pallas as pl

_SLEN = 512
_PT = 16
_STEP = 8
_NB = _SLEN // _STEP
_NPD = (_SLEN - _PT) // _STEP + 1
_HID = 64
_ODIM = 32


def _enc_kernel(blk_ref, q_ref, b1_ref, w2_ref, b2_ref, out_ref):
    blk = blk_ref[0]
    p = jnp.dot(blk, q_ref[...], preferred_element_type=jnp.float32)
    x = p[:, 0:64] + jnp.roll(p[:, 128:192], -64, axis=0)
    y = p[:, 64:128] + jnp.roll(p[:, 192:256], -64, axis=0)
    h = jnp.maximum(x + jnp.roll(y, -1, axis=0) + b1_ref[...], 0.0)
    o = jnp.dot(h.astype(jnp.bfloat16), w2_ref[...],
                preferred_element_type=jnp.float32) + b2_ref[...]
    out_ref[0] = o.reshape(_NB, _NB, _ODIM)[0:_NPD, 0:_NPD, :]


def kernel(image, W1, b1, W2, b2):
    B = image.shape[0]
    blk = (
        image.astype(jnp.bfloat16)
        .reshape(B, _NB, _STEP, _NB, _STEP)
        .transpose(0, 1, 3, 2, 4)
        .reshape(B, _NB * _NB, _STEP * _STEP)
    )
    w1r = W1.reshape(_PT, _PT, _HID)
    q = jnp.concatenate(
        [
            w1r[0:8, 0:8].reshape(64, _HID),
            w1r[0:8, 8:16].reshape(64, _HID),
            w1r[8:16, 0:8].reshape(64, _HID),
            w1r[8:16, 8:16].reshape(64, _HID),
        ],
        axis=1,
    ).astype(jnp.bfloat16)

    out = pl.pallas_call(
        _enc_kernel,
        grid=(B,),
        in_specs=[
            pl.BlockSpec((1, _NB * _NB, 64), lambda b: (b, 0, 0)),
            pl.BlockSpec((64, 4 * _HID), lambda b: (0, 0)),
            pl.BlockSpec((1, _HID), lambda b: (0, 0)),
            pl.BlockSpec((_HID, _ODIM), lambda b: (0, 0)),
            pl.BlockSpec((1, _ODIM), lambda b: (0, 0)),
        ],
        out_specs=pl.BlockSpec((1, _NPD, _NPD, _ODIM), lambda b: (b, 0, 0, 0)),
        out_shape=jax.ShapeDtypeStruct((B, _NPD, _NPD, _ODIM), jnp.float32),
    )(blk, q, b1.reshape(1, _HID), W2.astype(jnp.bfloat16), b2.reshape(1, _ODIM))

    return out.reshape(B * _NPD * _NPD, _ODIM)
